# baseline (speedup 1.0000x reference)
import functools

import jax
import jax.numpy as jnp
from jax import lax
from jax.experimental import pallas as pl
from jax.experimental.pallas import tpu as pltpu
from jax.experimental.pallas import tpu_sc as plsc

G = 2
V = 320
GV = G * V
DG = 128
TOKENS_PER_BLOCK = 512


def _tc_body(wt_ref, x_ref, b_ref, idx0_ref, idx1_ref, perp_ref, acc_ref):
    i = pl.program_id(0)
    n = pl.num_programs(0)
    T = x_ref.shape[0]

    logits_t = (
        lax.dot_general(
            wt_ref[...],
            x_ref[...],
            (((1,), (1,)), ((), ())),
            preferred_element_type=jnp.float32,
        )
        + b_ref[...]
    )

    row = lax.broadcasted_iota(jnp.int32, (V, T), 0)
    big = jnp.int32(1 << 30)
    l0 = logits_t[:V, :]
    l1 = logits_t[V:, :]
    m0 = jnp.max(l0, axis=0, keepdims=True)
    m1 = jnp.max(l1, axis=0, keepdims=True)
    i0 = jnp.min(jnp.where(l0 == m0, row, big), axis=0, keepdims=True)
    i1 = jnp.min(jnp.where(l1 == m1, row + V, big), axis=0, keepdims=True)

    idx0_ref[...] = i0.reshape(1, 1, T)
    idx1_ref[...] = i1.reshape(1, 1, T)

    rowg = lax.broadcasted_iota(jnp.int32, (GV, T), 0)
    onehot = ((rowg == i0) | (rowg == i1)).astype(jnp.float32)
    cnt = lax.dot_general(
        onehot,
        jnp.ones((T, 8), jnp.float32),
        (((1,), (0,)), ((), ())),
        preferred_element_type=jnp.float32,
    )

    @pl.when(i == 0)
    def _init():
        acc_ref[...] = jnp.zeros_like(acc_ref)

    acc_ref[...] += cnt

    @pl.when(i == n - 1)
    def _finalize():
        total = jnp.float32(n * T)
        p = acc_ref[...] / total
        e = p * jnp.log(p + 1e-7)
        h0 = jnp.sum(e[:V, 0:1])
        h1 = jnp.sum(e[V:, 0:1])
        perp_ref[...] = (jnp.exp(-h0) + jnp.exp(-h1)).reshape(1, 1)


def _tc_call(wt, x, b2d, interpret=False):
    nt = x.shape[0]
    nblk = nt // TOKENS_PER_BLOCK
    return pl.pallas_call(
        _tc_body,
        grid=(nblk,),
        in_specs=[
            pl.BlockSpec(wt.shape, lambda i: (0, 0)),
            pl.BlockSpec((TOKENS_PER_BLOCK, x.shape[1]), lambda i: (i, 0)),
            pl.BlockSpec(b2d.shape, lambda i: (0, 0)),
        ],
        out_specs=[
            pl.BlockSpec((1, 1, TOKENS_PER_BLOCK), lambda i: (i, 0, 0)),
            pl.BlockSpec((1, 1, TOKENS_PER_BLOCK), lambda i: (i, 0, 0)),
            pl.BlockSpec((1, 1), lambda i: (0, 0)),
        ],
        out_shape=[
            jax.ShapeDtypeStruct((nblk, 1, TOKENS_PER_BLOCK), jnp.int32),
            jax.ShapeDtypeStruct((nblk, 1, TOKENS_PER_BLOCK), jnp.int32),
            jax.ShapeDtypeStruct((1, 1), jnp.float32),
        ],
        scratch_shapes=[pltpu.VMEM((GV, 8), jnp.float32)],
        interpret=interpret,
    )(wt, x, b2d)


def _make_sc_gather(n_rows):
    info = plsc.get_sparse_core_info()
    nw = info.num_cores * info.num_subcores
    rows_per_w = n_rows // nw
    chunks = rows_per_w // 128
    mesh = plsc.VectorSubcoreMesh(core_axis_name="c", subcore_axis_name="s")

    @functools.partial(
        pl.kernel,
        out_type=jax.ShapeDtypeStruct((n_rows, DG), jnp.float32),
        mesh=mesh,
        scratch_types=[
            pltpu.VMEM((chunks, 128), jnp.int32),
            pltpu.VMEM((rows_per_w, DG), jnp.float32),
            pltpu.SemaphoreType.DMA,
        ],
    )
    def sc_gather(cb_hbm, idx_hbm, out_hbm, idx_v, rows_v, sem):
        wid = lax.axis_index("s") * info.num_cores + lax.axis_index("c")
        pltpu.sync_copy(idx_hbm.at[pl.ds(wid * chunks, chunks)], idx_v)
        copies = [
            pltpu.async_copy(
                cb_hbm.at[idx_v.at[j]], rows_v.at[pl.ds(j * 128, 128)], sem
            )
            for j in range(chunks)
        ]
        for c in copies:
            c.wait()
        pltpu.sync_copy(rows_v, out_hbm.at[pl.ds(wid * rows_per_w, rows_per_w)])

    return sc_gather


def kernel(hidden_states, W, b, codevectors):
    bsz, seq, hid = hidden_states.shape
    nt = bsz * seq
    x = hidden_states.reshape(nt, hid)

    idx0, idx1, perp = _tc_call(W.T, x, b.reshape(GV, 1))

    inter = jnp.stack([idx0.reshape(nt), idx1.reshape(nt)], axis=-1).reshape(
        nt * G
    )
    cb = codevectors.reshape(GV, DG)
    rows = _make_sc_gather(nt * G)(cb, inter.reshape(nt * G // 128, 128))
    cv = rows.reshape(bsz, seq, G * DG)
    return cv, perp.reshape(())


# speedup vs baseline: 1.2496x; 1.2496x over previous
import functools

import jax
import jax.numpy as jnp
from jax import lax
from jax.experimental import pallas as pl
from jax.experimental.pallas import tpu as pltpu
from jax.experimental.pallas import tpu_sc as plsc

G = 2
V = 320
GV = G * V
DG = 128
TOKENS_PER_BLOCK = 512


def _tc_body(w_ref, x_ref, b_ref, idx0_ref, idx1_ref, perp_ref, acc_ref):
    i = pl.program_id(0)
    n = pl.num_programs(0)
    T = x_ref.shape[1]

    logits_t = (
        lax.dot_general(
            w_ref[...],
            x_ref[0],
            (((0,), (1,)), ((), ())),
            preferred_element_type=jnp.float32,
        )
        + b_ref[...]
    )

    row = lax.broadcasted_iota(jnp.int32, (V, T), 0)
    big = jnp.int32(1 << 30)
    l0 = logits_t[:V, :]
    l1 = logits_t[V:, :]
    m0 = jnp.max(l0, axis=0, keepdims=True)
    m1 = jnp.max(l1, axis=0, keepdims=True)
    i0 = jnp.min(jnp.where(l0 == m0, row, big), axis=0, keepdims=True)
    i1 = jnp.min(jnp.where(l1 == m1, row + V, big), axis=0, keepdims=True)

    idx0_ref[...] = i0.reshape(1, 1, T)
    idx1_ref[...] = i1.reshape(1, 1, T)

    rowg = lax.broadcasted_iota(jnp.int32, (GV, T), 0)
    onehot = ((rowg == i0) | (rowg == i1)).astype(jnp.float32)
    cnt = lax.dot_general(
        onehot,
        jnp.ones((T, 8), jnp.float32),
        (((1,), (0,)), ((), ())),
        preferred_element_type=jnp.float32,
    )

    @pl.when(i == 0)
    def _init():
        acc_ref[...] = jnp.zeros_like(acc_ref)

    acc_ref[...] += cnt

    @pl.when(i == n - 1)
    def _finalize():
        total = jnp.float32(n * T)
        p = acc_ref[...] / total
        e = p * jnp.log(p + 1e-7)
        h0 = jnp.sum(e[:V, 0:1])
        h1 = jnp.sum(e[V:, 0:1])
        perp_ref[...] = (jnp.exp(-h0) + jnp.exp(-h1)).reshape(1, 1)


def _tc_call(w, hs, b2d, interpret=False):
    bsz, seq, hid = hs.shape
    nblk = (bsz * seq) // TOKENS_PER_BLOCK
    blocks_per_b = seq // TOKENS_PER_BLOCK
    return pl.pallas_call(
        _tc_body,
        grid=(nblk,),
        in_specs=[
            pl.BlockSpec(w.shape, lambda i: (0, 0)),
            pl.BlockSpec(
                (1, TOKENS_PER_BLOCK, hid),
                lambda i, _bp=blocks_per_b: (i // _bp, i % _bp, 0),
            ),
            pl.BlockSpec(b2d.shape, lambda i: (0, 0)),
        ],
        out_specs=[
            pl.BlockSpec((1, 1, TOKENS_PER_BLOCK), lambda i: (i, 0, 0)),
            pl.BlockSpec((1, 1, TOKENS_PER_BLOCK), lambda i: (i, 0, 0)),
            pl.BlockSpec((1, 1), lambda i: (0, 0)),
        ],
        out_shape=[
            jax.ShapeDtypeStruct((nblk, 1, TOKENS_PER_BLOCK), jnp.int32),
            jax.ShapeDtypeStruct((nblk, 1, TOKENS_PER_BLOCK), jnp.int32),
            jax.ShapeDtypeStruct((1, 1), jnp.float32),
        ],
        scratch_shapes=[pltpu.VMEM((GV, 8), jnp.float32)],
        interpret=interpret,
    )(w, hs, b2d)


def _make_sc_gather(bsz, seq):
    info = plsc.get_sparse_core_info()
    nw = info.num_cores * info.num_subcores
    tok_per_w = (bsz * seq) // nw
    w_per_b = seq // tok_per_w
    mesh = plsc.VectorSubcoreMesh(core_axis_name="c", subcore_axis_name="s")

    @functools.partial(
        pl.kernel,
        out_type=jax.ShapeDtypeStruct((bsz, seq, G * DG), jnp.float32),
        mesh=mesh,
        scratch_types=[
            pltpu.VMEM((tok_per_w,), jnp.int32),
            pltpu.VMEM((tok_per_w,), jnp.int32),
            pltpu.VMEM((tok_per_w, DG), jnp.float32),
            pltpu.VMEM((tok_per_w, DG), jnp.float32),
            pltpu.SemaphoreType.DMA,
        ],
    )
    def sc_gather(cb_hbm, idx0_hbm, idx1_hbm, out_hbm, i0_v, i1_v, r0_v, r1_v, sem):
        wid = lax.axis_index("s") * info.num_cores + lax.axis_index("c")
        brow = wid // w_per_b
        s0 = (wid % w_per_b) * tok_per_w
        pltpu.sync_copy(idx0_hbm.at[brow, 0, pl.ds(s0, tok_per_w)], i0_v)
        pltpu.sync_copy(idx1_hbm.at[brow, 0, pl.ds(s0, tok_per_w)], i1_v)
        c0 = pltpu.async_copy(cb_hbm.at[i0_v], r0_v, sem)
        c1 = pltpu.async_copy(cb_hbm.at[i1_v], r1_v, sem)
        c0.wait()
        c1.wait()
        pltpu.sync_copy(r0_v, out_hbm.at[brow, pl.ds(s0, tok_per_w), pl.ds(0, DG)])
        pltpu.sync_copy(r1_v, out_hbm.at[brow, pl.ds(s0, tok_per_w), pl.ds(DG, DG)])

    return sc_gather


def kernel(hidden_states, W, b, codevectors):
    bsz, seq, hid = hidden_states.shape

    idx0, idx1, perp = _tc_call(W, hidden_states, b.reshape(GV, 1))

    cb = codevectors.reshape(GV, DG)
    cv = _make_sc_gather(bsz, seq)(cb, idx0, idx1)
    return cv, perp.reshape(())


# speedup vs baseline: 1.3318x; 1.0657x over previous
import functools

import jax
import jax.numpy as jnp
from jax import lax
from jax.experimental import pallas as pl
from jax.experimental.pallas import tpu as pltpu
from jax.experimental.pallas import tpu_sc as plsc

G = 2
V = 320
GV = G * V
DG = 128
TOKENS_PER_BLOCK = 512


def _tc_body(w_ref, x_ref, b_ref, idx0_ref, idx1_ref, perp_ref, acc_ref):
    i = pl.program_id(0)
    n = pl.num_programs(0)
    rows_blk, seq_blk, hid = x_ref.shape
    T = rows_blk * seq_blk

    b_col = lax.dot_general(
        b_ref[...],
        jnp.ones((1, 1), jnp.float32),
        (((0,), (0,)), ((), ())),
        preferred_element_type=jnp.float32,
    )

    logits_t = (
        lax.dot_general(
            w_ref[...],
            x_ref[...].reshape(T, hid),
            (((0,), (1,)), ((), ())),
            preferred_element_type=jnp.float32,
        )
        + b_col
    )

    row = lax.broadcasted_iota(jnp.int32, (V, T), 0)
    big = jnp.int32(1 << 30)
    l0 = logits_t[:V, :]
    l1 = logits_t[V:, :]
    m0 = jnp.max(l0, axis=0, keepdims=True)
    m1 = jnp.max(l1, axis=0, keepdims=True)
    i0 = jnp.min(jnp.where(l0 == m0, row, big), axis=0, keepdims=True)
    i1 = jnp.min(jnp.where(l1 == m1, row + V, big), axis=0, keepdims=True)

    idx0_ref[...] = i0.reshape(1, 1, T)
    idx1_ref[...] = i1.reshape(1, 1, T)

    rowg = lax.broadcasted_iota(jnp.int32, (GV, T), 0)
    onehot = ((rowg == i0) | (rowg == i1)).astype(jnp.float32)
    cnt = lax.dot_general(
        onehot,
        jnp.ones((T, 8), jnp.float32),
        (((1,), (0,)), ((), ())),
        preferred_element_type=jnp.float32,
    )

    @pl.when(i == 0)
    def _init():
        acc_ref[...] = jnp.zeros_like(acc_ref)

    acc_ref[...] += cnt

    @pl.when(i == n - 1)
    def _finalize():
        total = jnp.float32(n * T)
        p = acc_ref[...] / total
        e = p * jnp.log(p + 1e-7)
        h0 = jnp.sum(e[:V, 0:1])
        h1 = jnp.sum(e[V:, 0:1])
        perp_ref[...] = (jnp.exp(-h0) + jnp.exp(-h1)).reshape(1, 1)


ROWS_PER_BLOCK = 2


def _tc_call(w, hs, b2d, interpret=False):
    bsz, seq, hid = hs.shape
    tpb = ROWS_PER_BLOCK * seq
    nblk = bsz // ROWS_PER_BLOCK
    return pl.pallas_call(
        _tc_body,
        grid=(nblk,),
        in_specs=[
            pl.BlockSpec(w.shape, lambda i: (0, 0)),
            pl.BlockSpec((ROWS_PER_BLOCK, seq, hid), lambda i: (i, 0, 0)),
            pl.BlockSpec(b2d.shape, lambda i: (0, 0)),
        ],
        out_specs=[
            pl.BlockSpec((1, 1, tpb), lambda i: (i, 0, 0)),
            pl.BlockSpec((1, 1, tpb), lambda i: (i, 0, 0)),
            pl.BlockSpec((1, 1), lambda i: (0, 0)),
        ],
        out_shape=[
            jax.ShapeDtypeStruct((nblk, 1, tpb), jnp.int32),
            jax.ShapeDtypeStruct((nblk, 1, tpb), jnp.int32),
            jax.ShapeDtypeStruct((1, 1), jnp.float32),
        ],
        scratch_shapes=[pltpu.VMEM((GV, 8), jnp.float32)],
        interpret=interpret,
    )(w, hs, b2d)


def _make_sc_gather(bsz, seq, idx_rows):
    info = plsc.get_sparse_core_info()
    nw = info.num_cores * info.num_subcores
    tok_per_w = (bsz * seq) // nw
    w_per_b = seq // tok_per_w
    idx_cols = (bsz * seq) // idx_rows
    w_per_i = idx_cols // tok_per_w
    mesh = plsc.VectorSubcoreMesh(core_axis_name="c", subcore_axis_name="s")

    @functools.partial(
        pl.kernel,
        out_type=jax.ShapeDtypeStruct((bsz, seq, G * DG), jnp.float32),
        mesh=mesh,
        scratch_types=[
            pltpu.VMEM((tok_per_w,), jnp.int32),
            pltpu.VMEM((tok_per_w,), jnp.int32),
            pltpu.VMEM((tok_per_w, DG), jnp.float32),
            pltpu.VMEM((tok_per_w, DG), jnp.float32),
            pltpu.SemaphoreType.DMA,
        ],
    )
    def sc_gather(cb_hbm, idx0_hbm, idx1_hbm, out_hbm, i0_v, i1_v, r0_v, r1_v, sem):
        wid = lax.axis_index("s") * info.num_cores + lax.axis_index("c")
        irow = wid // w_per_i
        ioff = (wid % w_per_i) * tok_per_w
        brow = wid // w_per_b
        s0 = (wid % w_per_b) * tok_per_w
        pltpu.sync_copy(idx0_hbm.at[irow, 0, pl.ds(ioff, tok_per_w)], i0_v)
        pltpu.sync_copy(idx1_hbm.at[irow, 0, pl.ds(ioff, tok_per_w)], i1_v)
        c0 = pltpu.async_copy(cb_hbm.at[i0_v], r0_v, sem)
        c1 = pltpu.async_copy(cb_hbm.at[i1_v], r1_v, sem)
        c0.wait()
        c1.wait()
        pltpu.sync_copy(r0_v, out_hbm.at[brow, pl.ds(s0, tok_per_w), pl.ds(0, DG)])
        pltpu.sync_copy(r1_v, out_hbm.at[brow, pl.ds(s0, tok_per_w), pl.ds(DG, DG)])

    return sc_gather


def kernel(hidden_states, W, b, codevectors):
    bsz, seq, hid = hidden_states.shape

    idx0, idx1, perp = _tc_call(W, hidden_states, b.reshape(1, GV))

    cb = codevectors.reshape(GV, DG)
    cv = _make_sc_gather(bsz, seq, idx0.shape[0])(cb, idx0, idx1)
    return cv, perp.reshape(())


# speedup vs baseline: 1.3930x; 1.0460x over previous
import functools

import jax
import jax.numpy as jnp
from jax import lax
from jax.experimental import pallas as pl
from jax.experimental.pallas import tpu as pltpu
from jax.experimental.pallas import tpu_sc as plsc

G = 2
V = 320
GV = G * V
DG = 128
TOKENS_PER_BLOCK = 512


def _tc_body(w_ref, x_ref, b_ref, idx0_ref, idx1_ref, perp_ref, acc_ref):
    i = pl.program_id(0)
    n = pl.num_programs(0)
    rows_blk, seq_blk, hid = x_ref.shape
    T = rows_blk * seq_blk

    b_col = lax.dot_general(
        b_ref[...].reshape(1, GV),
        jnp.ones((1, 1), jnp.float32),
        (((0,), (0,)), ((), ())),
        preferred_element_type=jnp.float32,
    )

    logits_t = (
        lax.dot_general(
            w_ref[...],
            x_ref[...].reshape(T, hid),
            (((0,), (1,)), ((), ())),
            preferred_element_type=jnp.float32,
        )
        + b_col
    )

    row = lax.broadcasted_iota(jnp.int32, (V, T), 0)
    big = jnp.int32(1 << 30)
    l0 = logits_t[:V, :]
    l1 = logits_t[V:, :]
    m0 = jnp.max(l0, axis=0, keepdims=True)
    m1 = jnp.max(l1, axis=0, keepdims=True)
    i0 = jnp.min(jnp.where(l0 == m0, row, big), axis=0, keepdims=True)
    i1 = jnp.min(jnp.where(l1 == m1, row + V, big), axis=0, keepdims=True)

    idx0_ref[...] = i0.reshape(1, 1, T)
    idx1_ref[...] = i1.reshape(1, 1, T)

    rowg = lax.broadcasted_iota(jnp.int32, (GV, T), 0)
    onehot = ((rowg == i0) | (rowg == i1)).astype(jnp.float32)
    cnt = lax.dot_general(
        onehot,
        jnp.ones((T, 8), jnp.float32),
        (((1,), (0,)), ((), ())),
        preferred_element_type=jnp.float32,
    )

    @pl.when(i == 0)
    def _init():
        acc_ref[...] = jnp.zeros_like(acc_ref)

    acc_ref[...] += cnt

    @pl.when(i == n - 1)
    def _finalize():
        total = jnp.float32(n * T)
        p = acc_ref[...] / total
        e = p * jnp.log(p + 1e-7)
        h0 = jnp.sum(e[:V, 0:1])
        h1 = jnp.sum(e[V:, 0:1])
        perp_ref[...] = (jnp.exp(-h0) + jnp.exp(-h1)).reshape(1, 1)


ROWS_PER_BLOCK = 2


def _tc_call(w, hs, b2d, interpret=False):
    bsz, seq, hid = hs.shape
    tpb = ROWS_PER_BLOCK * seq
    nblk = bsz // ROWS_PER_BLOCK
    return pl.pallas_call(
        _tc_body,
        grid=(nblk,),
        in_specs=[
            pl.BlockSpec(w.shape, lambda i: (0, 0)),
            pl.BlockSpec((ROWS_PER_BLOCK, seq, hid), lambda i: (i, 0, 0)),
            pl.BlockSpec(b2d.shape, lambda i: (0,)),
        ],
        out_specs=[
            pl.BlockSpec((1, 1, tpb), lambda i: (i, 0, 0)),
            pl.BlockSpec((1, 1, tpb), lambda i: (i, 0, 0)),
            pl.BlockSpec((1, 1), lambda i: (0, 0)),
        ],
        out_shape=[
            jax.ShapeDtypeStruct((nblk, 1, tpb), jnp.int32),
            jax.ShapeDtypeStruct((nblk, 1, tpb), jnp.int32),
            jax.ShapeDtypeStruct((1, 1), jnp.float32),
        ],
        scratch_shapes=[pltpu.VMEM((GV, 8), jnp.float32)],
        interpret=interpret,
    )(w, hs, b2d)


def _make_sc_gather(bsz, seq, idx_rows):
    info = plsc.get_sparse_core_info()
    nw = info.num_cores * info.num_subcores
    tok_per_w = (bsz * seq) // nw
    w_per_b = seq // tok_per_w
    idx_cols = (bsz * seq) // idx_rows
    w_per_i = idx_cols // tok_per_w
    mesh = plsc.VectorSubcoreMesh(core_axis_name="c", subcore_axis_name="s")

    @functools.partial(
        pl.kernel,
        out_type=jax.ShapeDtypeStruct((bsz, seq, G * DG), jnp.float32),
        mesh=mesh,
        scratch_types=[
            pltpu.VMEM((tok_per_w,), jnp.int32),
            pltpu.VMEM((tok_per_w,), jnp.int32),
            pltpu.VMEM((tok_per_w, DG), jnp.float32),
            pltpu.VMEM((tok_per_w, DG), jnp.float32),
            pltpu.SemaphoreType.DMA,
            pltpu.SemaphoreType.DMA,
        ],
    )
    def sc_gather(cb3_hbm, idx0_hbm, idx1_hbm, out_hbm, i0_v, i1_v, r0_v, r1_v, gsem, wsem):
        cb_hbm = cb3_hbm.at[0]
        wid = lax.axis_index("s") * info.num_cores + lax.axis_index("c")
        irow = wid // w_per_i
        ioff = (wid % w_per_i) * tok_per_w
        brow = wid // w_per_b
        s0 = (wid % w_per_b) * tok_per_w
        pltpu.sync_copy(idx0_hbm.at[irow, 0, pl.ds(ioff, tok_per_w)], i0_v)
        c0 = pltpu.async_copy(cb_hbm.at[i0_v], r0_v, gsem)
        pltpu.sync_copy(idx1_hbm.at[irow, 0, pl.ds(ioff, tok_per_w)], i1_v)
        c1 = pltpu.async_copy(cb_hbm.at[i1_v], r1_v, gsem)
        c0.wait()
        w0 = pltpu.async_copy(
            r0_v, out_hbm.at[brow, pl.ds(s0, tok_per_w), pl.ds(0, DG)], wsem
        )
        c1.wait()
        w1 = pltpu.async_copy(
            r1_v, out_hbm.at[brow, pl.ds(s0, tok_per_w), pl.ds(DG, DG)], wsem
        )
        w0.wait()
        w1.wait()

    return sc_gather


def kernel(hidden_states, W, b, codevectors):
    bsz, seq, hid = hidden_states.shape

    idx0, idx1, perp = _tc_call(W, hidden_states, b)

    cv = _make_sc_gather(bsz, seq, idx0.shape[0])(codevectors, idx0, idx1)
    return cv, perp.reshape(())


# speedup vs baseline: 1.4172x; 1.0174x over previous
import functools

import jax
import jax.numpy as jnp
from jax import lax
from jax.experimental import pallas as pl
from jax.experimental.pallas import tpu as pltpu
from jax.experimental.pallas import tpu_sc as plsc

G = 2
V = 320
GV = G * V
DG = 128
TOKENS_PER_BLOCK = 512


def _tc_body(w_ref, x_ref, b_ref, idx0_ref, idx1_ref, perp_ref, acc_ref):
    i = pl.program_id(0)
    n = pl.num_programs(0)
    rows_blk, seq_blk, hid = x_ref.shape
    T = rows_blk * seq_blk

    b_col = lax.dot_general(
        b_ref[...].reshape(1, GV),
        jnp.ones((1, 1), jnp.float32),
        (((0,), (0,)), ((), ())),
        preferred_element_type=jnp.float32,
    )

    logits_t = (
        lax.dot_general(
            w_ref[...],
            x_ref[...].reshape(T, hid),
            (((0,), (1,)), ((), ())),
            preferred_element_type=jnp.float32,
        )
        + b_col
    )

    row = lax.broadcasted_iota(jnp.int32, (V, T), 0)
    big = jnp.int32(1 << 30)
    l0 = logits_t[:V, :]
    l1 = logits_t[V:, :]
    m0 = jnp.max(l0, axis=0, keepdims=True)
    m1 = jnp.max(l1, axis=0, keepdims=True)
    i0 = jnp.min(jnp.where(l0 == m0, row, big), axis=0, keepdims=True)
    i1 = jnp.min(jnp.where(l1 == m1, row + V, big), axis=0, keepdims=True)

    idx0_ref[...] = i0.reshape(1, 1, T)
    idx1_ref[...] = i1.reshape(1, 1, T)

    rowg = lax.broadcasted_iota(jnp.int32, (GV, T), 0)
    onehot = ((rowg == i0) | (rowg == i1)).astype(jnp.float32)
    cnt = lax.dot_general(
        onehot,
        jnp.ones((T, 8), jnp.float32),
        (((1,), (0,)), ((), ())),
        preferred_element_type=jnp.float32,
    )

    @pl.when(i == 0)
    def _init():
        acc_ref[...] = jnp.zeros_like(acc_ref)

    acc_ref[...] += cnt

    @pl.when(i == n - 1)
    def _finalize():
        total = jnp.float32(n * T)
        p = acc_ref[...] / total
        e = p * jnp.log(p + 1e-7)
        h0 = jnp.sum(e[:V, 0:1])
        h1 = jnp.sum(e[V:, 0:1])
        perp_ref[...] = (jnp.exp(-h0) + jnp.exp(-h1)).reshape(1, 1)


ROWS_PER_BLOCK = 4


def _tc_call(w, hs, b2d, interpret=False):
    bsz, seq, hid = hs.shape
    tpb = ROWS_PER_BLOCK * seq
    nblk = bsz // ROWS_PER_BLOCK
    return pl.pallas_call(
        _tc_body,
        grid=(nblk,),
        in_specs=[
            pl.BlockSpec(w.shape, lambda i: (0, 0)),
            pl.BlockSpec((ROWS_PER_BLOCK, seq, hid), lambda i: (i, 0, 0)),
            pl.BlockSpec(b2d.shape, lambda i: (0,)),
        ],
        out_specs=[
            pl.BlockSpec((1, 1, tpb), lambda i: (i, 0, 0)),
            pl.BlockSpec((1, 1, tpb), lambda i: (i, 0, 0)),
            pl.BlockSpec((1, 1), lambda i: (0, 0)),
        ],
        out_shape=[
            jax.ShapeDtypeStruct((nblk, 1, tpb), jnp.int32),
            jax.ShapeDtypeStruct((nblk, 1, tpb), jnp.int32),
            jax.ShapeDtypeStruct((1, 1), jnp.float32),
        ],
        scratch_shapes=[pltpu.VMEM((GV, 8), jnp.float32)],
        interpret=interpret,
    )(w, hs, b2d)


def _make_sc_gather(bsz, seq, idx_rows):
    info = plsc.get_sparse_core_info()
    nw = info.num_cores * info.num_subcores
    tok_per_w = (bsz * seq) // nw
    w_per_b = seq // tok_per_w
    idx_cols = (bsz * seq) // idx_rows
    w_per_i = idx_cols // tok_per_w
    mesh = plsc.VectorSubcoreMesh(core_axis_name="c", subcore_axis_name="s")

    @functools.partial(
        pl.kernel,
        out_type=jax.ShapeDtypeStruct((bsz, seq, G * DG), jnp.float32),
        mesh=mesh,
        scratch_types=[
            pltpu.VMEM((tok_per_w,), jnp.int32),
            pltpu.VMEM((tok_per_w,), jnp.int32),
            pltpu.VMEM((tok_per_w, DG), jnp.float32),
            pltpu.VMEM((tok_per_w, DG), jnp.float32),
            pltpu.SemaphoreType.DMA,
            pltpu.SemaphoreType.DMA,
        ],
    )
    def sc_gather(cb3_hbm, idx0_hbm, idx1_hbm, out_hbm, i0_v, i1_v, r0_v, r1_v, gsem, wsem):
        cb_hbm = cb3_hbm.at[0]
        wid = lax.axis_index("s") * info.num_cores + lax.axis_index("c")
        irow = wid // w_per_i
        ioff = (wid % w_per_i) * tok_per_w
        brow = wid // w_per_b
        s0 = (wid % w_per_b) * tok_per_w
        pltpu.sync_copy(idx0_hbm.at[irow, 0, pl.ds(ioff, tok_per_w)], i0_v)
        c0 = pltpu.async_copy(cb_hbm.at[i0_v], r0_v, gsem)
        pltpu.sync_copy(idx1_hbm.at[irow, 0, pl.ds(ioff, tok_per_w)], i1_v)
        c1 = pltpu.async_copy(cb_hbm.at[i1_v], r1_v, gsem)
        c0.wait()
        w0 = pltpu.async_copy(
            r0_v, out_hbm.at[brow, pl.ds(s0, tok_per_w), pl.ds(0, DG)], wsem
        )
        c1.wait()
        w1 = pltpu.async_copy(
            r1_v, out_hbm.at[brow, pl.ds(s0, tok_per_w), pl.ds(DG, DG)], wsem
        )
        w0.wait()
        w1.wait()

    return sc_gather


def kernel(hidden_states, W, b, codevectors):
    bsz, seq, hid = hidden_states.shape

    idx0, idx1, perp = _tc_call(W, hidden_states, b)

    cv = _make_sc_gather(bsz, seq, idx0.shape[0])(codevectors, idx0, idx1)
    return cv, perp.reshape(())
